# baseline (speedup 1.0000x reference)
import functools

import jax
import jax.numpy as jnp
from jax import lax
from jax.experimental import pallas as pl
from jax.experimental.pallas import tpu as pltpu
from jax.experimental.pallas import tpu_sc as plsc

N = 10000
D = 128
B = 512
S = 256
NC = 2
NS = 16
NW = NC * NS

_mesh = plsc.VectorSubcoreMesh(core_axis_name="c", subcore_axis_name="s",
                               num_cores=NC, num_subcores=NS)


@functools.partial(
    pl.kernel,
    mesh=_mesh,
    out_type=[
        jax.ShapeDtypeStruct((B, D), jnp.float32),
        jax.ShapeDtypeStruct((S, D), jnp.float32),
        jax.ShapeDtypeStruct((B * S // 128, 128), jnp.float32),
    ],
    scratch_types=[
        pltpu.VMEM((16,), jnp.int32),
        pltpu.VMEM((16, D), jnp.float32),
        pltpu.VMEM((8,), jnp.int32),
        pltpu.VMEM((8, D), jnp.float32),
        pltpu.VMEM((2, 128), jnp.int32),
        pltpu.VMEM((2, 128), jnp.float32),
        pltpu.VMEM((2, 128), jnp.float32),
        pltpu.VMEM((32, 128), jnp.int32),
        pltpu.VMEM((32, 128), jnp.float32),
        pltpu.SemaphoreType.DMA,
    ],
)
def _sc_gather(feat_hbm, sup_hbm, p1_hbm, v_hbm, u_hbm, u2_hbm, sidx_hbm,
               xu1_hbm, xu0_hbm, so_hbm,
               vidx_v, vrows_v, uidx_v, urows_v, u2_v, p1u_v, rec_v,
               sidx_v, sup_v, sem):
    wid = lax.axis_index("s") * NC + lax.axis_index("c")

    base1 = wid * 16
    pltpu.sync_copy(v_hbm.at[pl.ds(base1, 16)], vidx_v)
    pltpu.async_copy(feat_hbm.at[vidx_v], vrows_v, sem).wait()
    pltpu.sync_copy(vrows_v, xu1_hbm.at[pl.ds(base1, 16)])

    base0 = wid * 8
    pltpu.sync_copy(u_hbm.at[pl.ds(base0, 8)], uidx_v)
    pltpu.async_copy(feat_hbm.at[uidx_v], urows_v, sem).wait()
    pltpu.sync_copy(urows_v, xu0_hbm.at[pl.ds(base0, 8)])

    pltpu.sync_copy(u2_hbm, u2_v)
    pltpu.async_copy(p1_hbm.at[u2_v.at[0]], p1u_v.at[0], sem).wait()
    pltpu.async_copy(p1_hbm.at[u2_v.at[1]], p1u_v.at[1], sem).wait()
    for cc in range(2):
        for kk in range(8):
            sl = pl.ds(kk * 16, 16)
            rec_v[cc, sl] = 1.0 / (p1u_v[cc, sl] * float(S))

    sbase = wid * 32
    pltpu.sync_copy(sidx_hbm.at[pl.ds(sbase, 32)], sidx_v)

    def gbody(j, carry):
        pltpu.async_copy(sup_hbm.at[sidx_v.at[j]], sup_v.at[j], sem).wait()
        return carry

    lax.fori_loop(0, 32, gbody, 0)

    def mbody(j, carry):
        par = lax.rem(j, 2)
        for kk in range(8):
            sl = pl.ds(kk * 16, 16)
            sup_v[j, sl] = sup_v[j, sl] * rec_v[par, sl]
        return carry

    lax.fori_loop(0, 32, mbody, 0)
    pltpu.sync_copy(sup_v, so_hbm.at[pl.ds(sbase, 32)])


def _loss_body(ft_ref, cs_ref, probs_ref, loss_ref):
    mask = (cs_ref[...] != 0.0).astype(jnp.float32)
    fm = ft_ref[...] * mask
    means = jnp.sum(fm, axis=0, keepdims=True)
    fc = fm - means
    lvec = jnp.sum(fc * fc * probs_ref[...], axis=0, keepdims=True)
    loss_ref[...] = jnp.sum(lvec, axis=1, keepdims=True) * (1.0 / D)


_loss_call = pl.pallas_call(
    _loss_body,
    in_specs=[
        pl.BlockSpec(memory_space=pltpu.MemorySpace.VMEM),
        pl.BlockSpec(memory_space=pltpu.MemorySpace.VMEM),
        pl.BlockSpec(memory_space=pltpu.MemorySpace.VMEM),
    ],
    out_specs=pl.BlockSpec(memory_space=pltpu.MemorySpace.VMEM),
    out_shape=jax.ShapeDtypeStruct((1, 1), jnp.float32),
)


def kernel(features, adj, w1, w2, v):
    support = adj[v, :]
    col_sums = jnp.sum(support, axis=0)
    num_neis = jnp.count_nonzero(col_sums)
    h_v = features[v]
    attention = h_v @ w1 + (features @ w2).reshape(1, -1) + 1.0
    attention = (1.0 / num_neis) * jax.nn.relu(attention)
    p1 = jnp.sum(support * attention, axis=0)
    probs = p1 / jnp.sum(p1)
    u = jax.random.choice(jax.random.key(42), N, shape=(S,), replace=True,
                          p=probs)

    u32 = u.astype(jnp.int32)
    sidx = (jnp.arange(B, dtype=jnp.int32)[:, None] * N
            + u32[None, :]).reshape(B * S // 128, 128)
    sup_flat = support.reshape(B * N)
    x_u1, x_u0, so = _sc_gather(features, sup_flat, p1, v.astype(jnp.int32),
                                u32, u32.reshape(2, 128), sidx)
    support_out = so.reshape(B, S)

    loss = _loss_call(features, col_sums.reshape(N, 1), probs.reshape(N, 1))
    return (x_u0, x_u1, support_out, loss[0, 0])


# speedup vs baseline: 1.1530x; 1.1530x over previous
import functools

import jax
import jax.numpy as jnp
from jax import lax
from jax.experimental import pallas as pl
from jax.experimental.pallas import tpu as pltpu
from jax.experimental.pallas import tpu_sc as plsc

N = 10000
D = 128
B = 512
S = 256
NC = 2
NS = 16
NW = NC * NS

_mesh = plsc.VectorSubcoreMesh(core_axis_name="c", subcore_axis_name="s",
                               num_cores=NC, num_subcores=NS)


@functools.partial(
    pl.kernel,
    mesh=_mesh,
    out_type=jax.ShapeDtypeStruct((B, D), jnp.float32),
    scratch_types=[
        pltpu.VMEM((16,), jnp.int32),
        pltpu.VMEM((16, D), jnp.float32),
        pltpu.SemaphoreType.DMA,
    ],
)
def _sc_xu1(feat_hbm, v_hbm, xu1_hbm, vidx_v, vrows_v, sem):
    wid = lax.axis_index("s") * NC + lax.axis_index("c")
    base1 = wid * 16
    pltpu.sync_copy(v_hbm.at[pl.ds(base1, 16)], vidx_v)
    pltpu.async_copy(feat_hbm.at[vidx_v], vrows_v, sem).wait()
    pltpu.sync_copy(vrows_v, xu1_hbm.at[pl.ds(base1, 16)])


@functools.partial(
    pl.kernel,
    mesh=_mesh,
    out_type=[
        jax.ShapeDtypeStruct((S, D), jnp.float32),
        jax.ShapeDtypeStruct((B * S // 128, 128), jnp.float32),
    ],
    scratch_types=[
        pltpu.VMEM((8,), jnp.int32),
        pltpu.VMEM((8, D), jnp.float32),
        pltpu.VMEM((2, 128), jnp.int32),
        pltpu.VMEM((2, 128), jnp.float32),
        pltpu.VMEM((2, 128), jnp.float32),
        pltpu.VMEM((32, 128), jnp.int32),
        pltpu.VMEM((32, 128), jnp.float32),
        pltpu.SemaphoreType.DMA,
        pltpu.SemaphoreType.DMA,
    ],
)
def _sc_gather(feat_hbm, sup_hbm, p1_hbm, u_hbm, u2_hbm, sidx_hbm,
               xu0_hbm, so_hbm,
               uidx_v, urows_v, u2_v, p1u_v, rec_v,
               sidx_v, sup_v, sem, gsem):
    wid = lax.axis_index("s") * NC + lax.axis_index("c")

    sbase = wid * 32
    pltpu.sync_copy(sidx_hbm.at[pl.ds(sbase, 32)], sidx_v)
    gcopies = [pltpu.make_async_copy(sup_hbm.at[sidx_v.at[j]], sup_v.at[j],
                                     gsem) for j in range(32)]
    for cp in gcopies:
        cp.start()

    base0 = wid * 8
    pltpu.sync_copy(u_hbm.at[pl.ds(base0, 8)], uidx_v)
    pltpu.async_copy(feat_hbm.at[uidx_v], urows_v, sem).wait()
    pltpu.sync_copy(urows_v, xu0_hbm.at[pl.ds(base0, 8)])

    pltpu.sync_copy(u2_hbm, u2_v)
    c0 = pltpu.make_async_copy(p1_hbm.at[u2_v.at[0]], p1u_v.at[0], sem)
    c1 = pltpu.make_async_copy(p1_hbm.at[u2_v.at[1]], p1u_v.at[1], sem)
    c0.start()
    c1.start()
    c0.wait()
    c1.wait()
    for cc in range(2):
        for kk in range(8):
            sl = pl.ds(kk * 16, 16)
            rec_v[cc, sl] = 1.0 / (p1u_v[cc, sl] * float(S))

    for cp in gcopies:
        cp.wait()

    def mbody(j, carry):
        par = lax.rem(j, 2)
        for kk in range(8):
            sl = pl.ds(kk * 16, 16)
            sup_v[j, sl] = sup_v[j, sl] * rec_v[par, sl]
        return carry

    lax.fori_loop(0, 32, mbody, 0)
    pltpu.sync_copy(sup_v, so_hbm.at[pl.ds(sbase, 32)])


def _loss_body(ft_ref, cs_ref, probs_ref, loss_ref):
    mask = (cs_ref[...] != 0.0).astype(jnp.float32)
    fm = ft_ref[...] * mask
    means = jnp.sum(fm, axis=0, keepdims=True)
    fc = fm - means
    lvec = jnp.sum(fc * fc * probs_ref[...], axis=0, keepdims=True)
    loss_ref[...] = jnp.sum(lvec, axis=1, keepdims=True) * (1.0 / D)


_loss_call = pl.pallas_call(
    _loss_body,
    in_specs=[
        pl.BlockSpec(memory_space=pltpu.MemorySpace.VMEM),
        pl.BlockSpec(memory_space=pltpu.MemorySpace.VMEM),
        pl.BlockSpec(memory_space=pltpu.MemorySpace.VMEM),
    ],
    out_specs=pl.BlockSpec(memory_space=pltpu.MemorySpace.VMEM),
    out_shape=jax.ShapeDtypeStruct((1, 1), jnp.float32),
)


def kernel(features, adj, w1, w2, v):
    support = adj[v, :]
    col_sums = jnp.sum(support, axis=0)
    num_neis = jnp.count_nonzero(col_sums)
    h_v = features[v]
    attention = h_v @ w1 + (features @ w2).reshape(1, -1) + 1.0
    attention = (1.0 / num_neis) * jax.nn.relu(attention)
    p1 = jnp.sum(support * attention, axis=0)
    probs = p1 / jnp.sum(p1)
    u = jax.random.choice(jax.random.key(42), N, shape=(S,), replace=True,
                          p=probs)

    u32 = u.astype(jnp.int32)
    sidx = (jnp.arange(B, dtype=jnp.int32)[:, None] * N
            + u32[None, :]).reshape(B * S // 128, 128)
    sup_flat = support.reshape(B * N)
    x_u1 = _sc_xu1(features, v.astype(jnp.int32))
    x_u0, so = _sc_gather(features, sup_flat, p1,
                          u32, u32.reshape(2, 128), sidx)
    support_out = so.reshape(B, S)

    loss = _loss_call(features, col_sums.reshape(N, 1), probs.reshape(N, 1))
    return (x_u0, x_u1, support_out, loss[0, 0])


# speedup vs baseline: 1.1965x; 1.0377x over previous
import functools

import jax
import jax.numpy as jnp
from jax import lax
from jax.experimental import pallas as pl
from jax.experimental.pallas import tpu as pltpu
from jax.experimental.pallas import tpu_sc as plsc

N = 10000
D = 128
B = 512
S = 256
NC = 2
NS = 16
NW = NC * NS

_mesh = plsc.VectorSubcoreMesh(core_axis_name="c", subcore_axis_name="s",
                               num_cores=NC, num_subcores=NS)


@functools.partial(
    pl.kernel,
    mesh=_mesh,
    out_type=jax.ShapeDtypeStruct((B, D), jnp.float32),
    scratch_types=[
        pltpu.VMEM((16,), jnp.int32),
        pltpu.VMEM((16, D), jnp.float32),
        pltpu.SemaphoreType.DMA,
    ],
)
def _sc_xu1(feat_hbm, v_hbm, xu1_hbm, vidx_v, vrows_v, sem):
    wid = lax.axis_index("s") * NC + lax.axis_index("c")
    base1 = wid * 16
    pltpu.sync_copy(v_hbm.at[pl.ds(base1, 16)], vidx_v)
    pltpu.async_copy(feat_hbm.at[vidx_v], vrows_v, sem).wait()
    pltpu.sync_copy(vrows_v, xu1_hbm.at[pl.ds(base1, 16)])


@functools.partial(
    pl.kernel,
    mesh=_mesh,
    out_type=[
        jax.ShapeDtypeStruct((S, D), jnp.float32),
        jax.ShapeDtypeStruct((B * S // 128, 128), jnp.float32),
    ],
    scratch_types=[
        pltpu.VMEM((8,), jnp.int32),
        pltpu.VMEM((8, D), jnp.float32),
        pltpu.VMEM((2, 128), jnp.int32),
        pltpu.VMEM((2, 128), jnp.float32),
        pltpu.VMEM((2, 128), jnp.float32),
        pltpu.VMEM((32, 128), jnp.int32),
        pltpu.VMEM((32, 128), jnp.float32),
        pltpu.SemaphoreType.DMA,
        pltpu.SemaphoreType.DMA,
    ],
)
def _sc_gather(feat_hbm, sup_hbm, p1_hbm, u_hbm, u2_hbm, sidx_hbm,
               xu0_hbm, so_hbm,
               uidx_v, urows_v, u2_v, p1u_v, rec_v,
               sidx_v, sup_v, sem, gsem):
    wid = lax.axis_index("s") * NC + lax.axis_index("c")

    sbase = wid * 32
    pltpu.sync_copy(sidx_hbm.at[pl.ds(sbase, 32)], sidx_v)
    gcopies = [pltpu.make_async_copy(sup_hbm.at[sidx_v.at[j]], sup_v.at[j],
                                     gsem) for j in range(32)]
    for cp in gcopies:
        cp.start()

    base0 = wid * 8
    pltpu.sync_copy(u_hbm.at[pl.ds(base0, 8)], uidx_v)
    pltpu.async_copy(feat_hbm.at[uidx_v], urows_v, sem).wait()
    pltpu.sync_copy(urows_v, xu0_hbm.at[pl.ds(base0, 8)])

    pltpu.sync_copy(u2_hbm, u2_v)
    c0 = pltpu.make_async_copy(p1_hbm.at[u2_v.at[0]], p1u_v.at[0], sem)
    c1 = pltpu.make_async_copy(p1_hbm.at[u2_v.at[1]], p1u_v.at[1], sem)
    c0.start()
    c1.start()
    c0.wait()
    c1.wait()
    for cc in range(2):
        for kk in range(8):
            sl = pl.ds(kk * 16, 16)
            rec_v[cc, sl] = 1.0 / (p1u_v[cc, sl] * float(S))

    for cp in gcopies:
        cp.wait()

    for j in range(32):
        for kk in range(8):
            sl = pl.ds(kk * 16, 16)
            sup_v[j, sl] = sup_v[j, sl] * rec_v[j % 2, sl]
    pltpu.sync_copy(sup_v, so_hbm.at[pl.ds(sbase, 32)])


def _loss_body(ft_ref, cs_ref, probs_ref, loss_ref):
    mask = (cs_ref[...] != 0.0).astype(jnp.float32)
    f = ft_ref[...]
    pr = probs_ref[...]
    means = jnp.dot(mask, f, preferred_element_type=jnp.float32)
    s1 = jnp.dot(pr, f, preferred_element_type=jnp.float32)
    s2 = jnp.dot(pr, f * f, preferred_element_type=jnp.float32)
    sp = jnp.sum(pr)
    lvec = s2 - 2.0 * means * s1 + means * means * sp
    loss_ref[...] = jnp.sum(lvec, axis=1, keepdims=True) * (1.0 / D)


_loss_call = pl.pallas_call(
    _loss_body,
    in_specs=[
        pl.BlockSpec(memory_space=pltpu.MemorySpace.VMEM),
        pl.BlockSpec(memory_space=pltpu.MemorySpace.VMEM),
        pl.BlockSpec(memory_space=pltpu.MemorySpace.VMEM),
    ],
    out_specs=pl.BlockSpec(memory_space=pltpu.MemorySpace.VMEM),
    out_shape=jax.ShapeDtypeStruct((1, 1), jnp.float32),
)


def kernel(features, adj, w1, w2, v):
    support = adj[v, :]
    col_sums = jnp.sum(support, axis=0)
    num_neis = jnp.count_nonzero(col_sums)
    h_v = features[v]
    attention = h_v @ w1 + (features @ w2).reshape(1, -1) + 1.0
    attention = (1.0 / num_neis) * jax.nn.relu(attention)
    p1 = jnp.sum(support * attention, axis=0)
    probs = p1 / jnp.sum(p1)
    u = jax.random.choice(jax.random.key(42), N, shape=(S,), replace=True,
                          p=probs)

    u32 = u.astype(jnp.int32)
    sidx = (jnp.arange(B, dtype=jnp.int32)[:, None] * N
            + u32[None, :]).reshape(B * S // 128, 128)
    sup_flat = support.reshape(B * N)
    x_u1 = _sc_xu1(features, v.astype(jnp.int32))
    x_u0, so = _sc_gather(features, sup_flat, p1,
                          u32, u32.reshape(2, 128), sidx)
    support_out = so.reshape(B, S)

    loss = _loss_call(features, col_sums.reshape(1, N), probs.reshape(1, N))
    return (x_u0, x_u1, support_out, loss[0, 0])
